# Initial kernel scaffold; baseline (speedup 1.0000x reference)
import jax
import jax.numpy as jnp
from jax.experimental import pallas as pl


def kernel(input_ids, embed_tokens, cos_cached, sin_cached):
    raise NotImplementedError("write your pallas kernel here")


# speedup vs baseline: 1.4893x; 1.4893x over previous
import functools

import jax
import jax.numpy as jnp
from jax import lax
from jax.experimental import pallas as pl
from jax.experimental.pallas import tpu as pltpu
from jax.experimental.pallas import tpu_sc as plsc

VOCAB = 100000
DIM = 1024
BATCH = 4
SEQ = 2048

_info = plsc.get_sparse_core_info()
NC, NS = _info.num_cores, _info.num_subcores
NW = NC * NS
TOTAL_ROWS = BATCH * SEQ
ROWS_PER_W = TOTAL_ROWS // NW
CHUNK = 32
N_CHUNKS = ROWS_PER_W // CHUNK


def _gather_body(idx_hbm, table_hbm, out_hbm, idx_v, buf0, buf1, sem_g0,
                 sem_g1, sem_o0, sem_o1):
    wid = lax.axis_index("s") * NC + lax.axis_index("c")
    base = wid * ROWS_PER_W
    pltpu.sync_copy(idx_hbm.at[wid], idx_v)

    bufs = (buf0, buf1)
    gsems = (sem_g0, sem_g1)
    osems = (sem_o0, sem_o1)

    g0 = pltpu.async_copy(table_hbm.at[idx_v.at[0]], bufs[0], gsems[0])
    pending_g = [g0, None]
    pending_o = [None, None]
    for c in range(N_CHUNKS):
        s = c % 2
        ns = (c + 1) % 2
        if c + 1 < N_CHUNKS:
            if pending_o[ns] is not None:
                pending_o[ns].wait()
                pending_o[ns] = None
            pending_g[ns] = pltpu.async_copy(
                table_hbm.at[idx_v.at[c + 1]], bufs[ns], gsems[ns])
        pending_g[s].wait()
        pending_o[s] = pltpu.async_copy(
            bufs[s], out_hbm.at[pl.ds(base + c * CHUNK, CHUNK)], osems[s])
    for o in pending_o:
        if o is not None:
            o.wait()


@functools.partial(jax.jit, donate_argnums=())
def _embedding_gather(ids_3d, table):
    mesh = plsc.VectorSubcoreMesh(core_axis_name="c", subcore_axis_name="s")
    kern = functools.partial(
        pl.kernel,
        mesh=mesh,
        out_type=jax.ShapeDtypeStruct((TOTAL_ROWS, DIM), jnp.float32),
        scratch_types=[
            pltpu.VMEM((N_CHUNKS, CHUNK), jnp.int32),
            pltpu.VMEM((CHUNK, DIM), jnp.float32),
            pltpu.VMEM((CHUNK, DIM), jnp.float32),
            pltpu.SemaphoreType.DMA,
            pltpu.SemaphoreType.DMA,
            pltpu.SemaphoreType.DMA,
            pltpu.SemaphoreType.DMA,
        ],
    )(_gather_body)
    return kern(ids_3d, table)


def kernel(input_ids, embed_tokens, cos_cached, sin_cached):
    seq_len = input_ids.shape[1]
    ids_3d = input_ids.reshape(NW, N_CHUNKS, CHUNK)
    rows = _embedding_gather(ids_3d, embed_tokens)
    x = rows.reshape(input_ids.shape[0], seq_len, DIM)
    cos = cos_cached[:seq_len].astype(x.dtype)
    sin = sin_cached[:seq_len].astype(x.dtype)
    return (x, cos, sin)
